# baseline (speedup 1.0000x reference)
import functools

import jax
import jax.numpy as jnp
from jax import lax
from jax.experimental import pallas as pl
from jax.experimental.pallas import tpu as pltpu
from jax.experimental.pallas import tpu_sc as plsc

NC = 2
NS = 16
NW = NC * NS


def _mesh():
    return plsc.VectorSubcoreMesh(core_axis_name="c", subcore_axis_name="s")


_SC_PARAMS = pltpu.CompilerParams(needs_layout_passes=False,
                                  use_tc_tiling_on_sc=False)


def _deg_kernel(n_pad, per_w):
    @functools.partial(
        pl.kernel,
        mesh=_mesh(),
        compiler_params=_SC_PARAMS,
        out_type=jax.ShapeDtypeStruct((NW, n_pad), jnp.float32),
        scratch_types=[
            pltpu.VMEM((per_w,), jnp.int32),
            pltpu.VMEM((n_pad,), jnp.float32),
        ],
    )
    def k(dst_hbm, zeros_hbm, deg_hbm, idx_v, hist_v):
        c = lax.axis_index("c")
        s = lax.axis_index("s")
        w = s * NC + c
        pltpu.sync_copy(dst_hbm.at[w], idx_v)
        pltpu.sync_copy(zeros_hbm.at[pl.ds(0, n_pad)], hist_v)
        ones = jnp.full((16,), 1.0, jnp.float32)

        def body(t, carry):
            v = idx_v[pl.ds(t * 16, 16)]
            plsc.addupdate_scatter(hist_v, [v], ones)
            return carry

        lax.fori_loop(0, per_w // 16, body, None)
        pltpu.sync_copy(hist_v, deg_hbm.at[w])

    return k


def _agg_kernel(n_pad, d, nb, b):
    rows = n_pad // NS

    @functools.partial(
        pl.kernel,
        mesh=_mesh(),
        compiler_params=_SC_PARAMS,
        out_type=jax.ShapeDtypeStruct((NC, n_pad, d), jnp.float32),
        scratch_types=[
            pltpu.VMEM((nb + 1, b), jnp.int32),
            pltpu.VMEM((nb, b), jnp.int32),
            pltpu.VMEM((b, d), jnp.float32),
            pltpu.VMEM((b, d), jnp.float32),
            pltpu.VMEM_SHARED((n_pad, d), jnp.float32),
            pltpu.SemaphoreType.DMA,
            pltpu.SemaphoreType.DMA,
        ],
    )
    def k(g_hbm, src_hbm, dst_hbm, zeros_hbm, out_hbm,
          src_v, dst_v, g0, g1, acc_sh, sem0, sem1):
        c = lax.axis_index("c")
        s = lax.axis_index("s")
        w = s * NC + c
        pltpu.sync_copy(src_hbm.at[w], src_v)
        pltpu.sync_copy(dst_hbm.at[w], dst_v)
        pltpu.sync_copy(zeros_hbm, acc_sh.at[pl.ds(s * rows, rows)])
        plsc.subcore_barrier()

        def body(j, carry):
            pltpu.async_copy(g_hbm.at[src_v.at[j]], g0, sem0).wait()
            pltpu.sync_copy(g0, acc_sh.at[dst_v.at[j]], add=True)
            return carry

        lax.fori_loop(0, nb, body, None)
        plsc.subcore_barrier()
        pltpu.sync_copy(acc_sh.at[pl.ds(s * rows, rows)],
                        out_hbm.at[c, pl.ds(s * rows, rows)])

    return k


def _decode_kernel(n_pad, nbd, ch):
    rows_core = 2 * nbd
    rows_tile = rows_core // NS

    @functools.partial(
        pl.kernel,
        mesh=_mesh(),
        compiler_params=_SC_PARAMS,
        out_type=jax.ShapeDtypeStruct((4 * nbd, ch), jnp.float32),
        scratch_types=[
            pltpu.VMEM((8, n_pad), jnp.float32),
            pltpu.VMEM((ch,), jnp.int32),
            pltpu.VMEM((ch,), jnp.int32),
            pltpu.VMEM((ch,), jnp.int32),
            pltpu.VMEM((ch,), jnp.int32),
            pltpu.VMEM((16, ch), jnp.float32),
            pltpu.VMEM((16, ch), jnp.float32),
            pltpu.VMEM((16,), jnp.int32),
            pltpu.VMEM((16,), jnp.int32),
            pltpu.VMEM_SHARED((rows_core, ch), jnp.float32),
            pltpu.SemaphoreType.DMA,
            pltpu.SemaphoreType.DMA,
            pltpu.SemaphoreType.DMA,
            pltpu.SemaphoreType.DMA,
            pltpu.SemaphoreType.DMA,
            pltpu.SemaphoreType.DMA,
        ],
    )
    def k(zt_hbm, src_hbm, dst_hbm, zeros_hbm, out_hbm,
          zv, sv0, dv0, sv1, dv1, pv0, pv1, ri0, ri1, acc_sh,
          ss0, sd0, ss1, sd1, sa0, sa1):
        c = lax.axis_index("c")
        s = lax.axis_index("s")
        fg = lax.rem(s, 8)
        egl = s // 8
        eg = c * 2 + egl
        pltpu.sync_copy(zt_hbm.at[pl.ds(fg * 8, 8)], zv)
        pltpu.sync_copy(zeros_hbm,
                        acc_sh.at[pl.ds(s * rows_tile, rows_tile)])
        plsc.subcore_barrier()
        fidx = [jnp.full((16,), f, jnp.int32) for f in range(8)]
        pltpu.async_copy(src_hbm.at[eg, 0], sv0, ss0)
        pltpu.async_copy(dst_hbm.at[eg, 0], dv0, sd0)

        def process(sv, dv, pv, row):
            def block(off):
                s16 = sv[pl.ds(off, 16)]
                t16 = dv[pl.ds(off, 16)]
                acc0 = jnp.zeros((16,), jnp.float32)
                acc1 = jnp.zeros((16,), jnp.float32)
                for f in range(4):
                    a = plsc.load_gather(zv, [fidx[f], s16])
                    bb = plsc.load_gather(zv, [fidx[f], t16])
                    acc0 = acc0 + a * bb
                for f in range(4, 8):
                    a = plsc.load_gather(zv, [fidx[f], s16])
                    bb = plsc.load_gather(zv, [fidx[f], t16])
                    acc1 = acc1 + a * bb
                pv[row, pl.ds(off, 16)] = acc0 + acc1

            def inner(i, carry2):
                block(i * 32)
                block(i * 32 + 16)
                return carry2

            lax.fori_loop(0, ch // 32, inner, None)

        def run_part(p, pv, ri, sa, drain_prev):
            if drain_prev:
                pltpu.make_async_copy(pv, acc_sh.at[ri], sa).wait()

            def chunk2(q2, carry1):
                kk = p * 16 + 2 * q2
                pltpu.async_copy(src_hbm.at[eg, kk + 1], sv1, ss1)
                pltpu.async_copy(dst_hbm.at[eg, kk + 1], dv1, sd1)
                pltpu.make_async_copy(src_hbm.at[eg, kk], sv0, ss0).wait()
                pltpu.make_async_copy(dst_hbm.at[eg, kk], dv0, sd0).wait()
                process(sv0, dv0, pv, 2 * q2)
                pltpu.async_copy(src_hbm.at[eg, kk + 2], sv0, ss0)
                pltpu.async_copy(dst_hbm.at[eg, kk + 2], dv0, sd0)
                pltpu.make_async_copy(src_hbm.at[eg, kk + 1], sv1, ss1).wait()
                pltpu.make_async_copy(dst_hbm.at[eg, kk + 1], dv1, sd1).wait()
                process(sv1, dv1, pv, 2 * q2 + 1)
                return carry1

            lax.fori_loop(0, 8, chunk2, None)
            ri[...] = lax.iota(jnp.int32, 16) + (egl * nbd + p * 16)
            pltpu.async_copy(pv, acc_sh.at[ri], sa, add=True)

        run_part(0, pv0, ri0, sa0, False)
        run_part(1, pv1, ri1, sa1, False)

        def part2(t, carry):
            run_part(2 * t, pv0, ri0, sa0, True)
            run_part(2 * t + 1, pv1, ri1, sa1, True)
            return carry

        lax.fori_loop(1, nbd // 32, part2, None)
        pltpu.make_async_copy(src_hbm.at[eg, nbd], sv0, ss0).wait()
        pltpu.make_async_copy(dst_hbm.at[eg, nbd], dv0, sd0).wait()
        pltpu.make_async_copy(pv0, acc_sh.at[ri0], sa0).wait()
        pltpu.make_async_copy(pv1, acc_sh.at[ri1], sa1).wait()
        plsc.subcore_barrier()
        pltpu.sync_copy(acc_sh.at[pl.ds(s * rows_tile, rows_tile)],
                        out_hbm.at[pl.ds(c * rows_core + s * rows_tile, rows_tile)])

    return k


def _tc_prep(n_pad, d_in, d_h, r):
    def body(deg2_ref, mask_ref, x_ref, w1_ref, h1_ref, g1_ref, dinv_ref):
        deg = jnp.sum(deg2_ref[...], axis=0) + mask_ref[...]
        dinv = jnp.where(deg > 0, lax.rsqrt(deg), 0.0)
        h1 = jnp.dot(x_ref[...], w1_ref[...], preferred_element_type=jnp.float32)
        h1_ref[...] = h1
        g1_ref[...] = h1 * dinv[:, None]
        dinv_ref[...] = dinv

    return pl.pallas_call(
        body,
        grid=(n_pad // r,),
        in_specs=[
            pl.BlockSpec((NW, r), lambda i: (0, i)),
            pl.BlockSpec((r,), lambda i: (i,)),
            pl.BlockSpec((r, d_in), lambda i: (i, 0)),
            pl.BlockSpec((d_in, d_h), lambda i: (0, 0)),
        ],
        out_specs=[
            pl.BlockSpec((r, d_h), lambda i: (i, 0)),
            pl.BlockSpec((r, d_h), lambda i: (i, 0)),
            pl.BlockSpec((r,), lambda i: (i,)),
        ],
        out_shape=[
            jax.ShapeDtypeStruct((n_pad, d_h), jnp.float32),
            jax.ShapeDtypeStruct((n_pad, d_h), jnp.float32),
            jax.ShapeDtypeStruct((n_pad,), jnp.float32),
        ],
    )


def _tc_mid(n_pad, d_h, d_out, r):
    def body(p_ref, h1_ref, dinv_ref, b1_ref, w2_ref, h2_ref, g2_ref):
        dinv = dinv_ref[...]
        z1 = (dinv[:, None] * (p_ref[0] + p_ref[1])
              + (dinv * dinv)[:, None] * h1_ref[...] + b1_ref[...][None, :])
        z1 = jnp.maximum(z1, 0.0)
        h2 = jnp.dot(z1, w2_ref[...], preferred_element_type=jnp.float32)
        h2_ref[...] = h2
        g2_ref[...] = h2 * dinv[:, None]

    return pl.pallas_call(
        body,
        grid=(n_pad // r,),
        in_specs=[
            pl.BlockSpec((2, r, d_h), lambda i: (0, i, 0)),
            pl.BlockSpec((r, d_h), lambda i: (i, 0)),
            pl.BlockSpec((r,), lambda i: (i,)),
            pl.BlockSpec((d_h,), lambda i: (0,)),
            pl.BlockSpec((d_h, d_out), lambda i: (0, 0)),
        ],
        out_specs=[
            pl.BlockSpec((r, d_out), lambda i: (i, 0)),
            pl.BlockSpec((r, d_out), lambda i: (i, 0)),
        ],
        out_shape=[
            jax.ShapeDtypeStruct((n_pad, d_out), jnp.float32),
            jax.ShapeDtypeStruct((n_pad, d_out), jnp.float32),
        ],
    )


def _tc_post(n_pad, d_out, r):
    def body(p_ref, h2_ref, dinv_ref, b2_ref, z2_ref):
        dinv = dinv_ref[...]
        z2_ref[...] = (dinv[:, None] * (p_ref[0] + p_ref[1])
                       + (dinv * dinv)[:, None] * h2_ref[...]
                       + b2_ref[...][None, :])

    return pl.pallas_call(
        body,
        grid=(n_pad // r,),
        in_specs=[
            pl.BlockSpec((2, r, d_out), lambda i: (0, i, 0)),
            pl.BlockSpec((r, d_out), lambda i: (i, 0)),
            pl.BlockSpec((r,), lambda i: (i,)),
            pl.BlockSpec((d_out,), lambda i: (0,)),
        ],
        out_specs=pl.BlockSpec((r, d_out), lambda i: (i, 0)),
        out_shape=jax.ShapeDtypeStruct((n_pad, d_out), jnp.float32),
    )


def kernel(x, pos_edge_index, neg_edge_index, W1, b1, W2, b2):
    n, d_in = x.shape
    d_h = W1.shape[1]
    d_out = W2.shape[1]
    e = pos_edge_index.shape[1]

    n_pad = ((n + NW * 8 - 1) // (NW * 8)) * (NW * 8)
    dummy = n

    spare = max(n_pad - n, 1)

    def spread_pad(idx, e_pad):
        pad = dummy + (jnp.arange(e_pad - e, dtype=idx.dtype) % spare)
        return jnp.concatenate([idx, pad])

    def agg_layout(b):
        nb = -(-e // (NW * b))
        nb = nb + (nb % 2)
        e_pad = NW * nb * b
        sp = spread_pad(pos_edge_index[0], e_pad).reshape(NW, nb, b)
        dp = spread_pad(pos_edge_index[1], e_pad).reshape(NW, nb, b)
        sp = jnp.concatenate(
            [sp, jnp.full((NW, 1, b), dummy, sp.dtype)], axis=1)
        return sp, dp, nb

    ba1 = 128
    ba2 = 128
    src1, dst1, nb1 = agg_layout(ba1)
    src2, dst2, nb2 = agg_layout(ba2)
    dst_flat = dst2.reshape(NW, nb2 * ba2)

    ch = 512
    e2 = 2 * e
    nbd = -(-e2 // (4 * ch))
    nbd = ((nbd + 31) // 32) * 32
    e2_pad = 4 * nbd * ch
    pad_d = dummy + (jnp.arange(e2_pad - e2, dtype=jnp.int32) % spare)
    src_d = jnp.concatenate(
        [pos_edge_index[0], neg_edge_index[0], pad_d]).reshape(4, nbd, ch)
    dst_d = jnp.concatenate(
        [pos_edge_index[1], neg_edge_index[1], pad_d]).reshape(4, nbd, ch)
    pad_chunk = jnp.full((4, 1, ch), dummy, jnp.int32)
    src_d = jnp.concatenate([src_d, pad_chunk], axis=1)
    dst_d = jnp.concatenate([dst_d, pad_chunk], axis=1)

    xp = jnp.zeros((n_pad, d_in), jnp.float32).at[:n].set(x)
    maskf = (jnp.arange(n_pad) < n).astype(jnp.float32)

    zeros1d = jnp.zeros((n_pad,), jnp.float32)
    zeros_h = jnp.zeros((n_pad // NS, d_h), jnp.float32)
    zeros_o = jnp.zeros((n_pad // NS, d_out), jnp.float32)
    zeros_d = jnp.zeros(((2 * nbd) // NS, ch), jnp.float32)

    r = 1024

    deg2 = _deg_kernel(n_pad, nb2 * ba2)(dst_flat, zeros1d)
    h1, g1, dinv = _tc_prep(n_pad, d_in, d_h, r)(deg2, maskf, xp, W1)
    part1 = _agg_kernel(n_pad, d_h, nb1, ba1)(g1, src1, dst1, zeros_h)
    h2, g2 = _tc_mid(n_pad, d_h, d_out, r)(part1, h1, dinv, b1, W2)
    part2 = _agg_kernel(n_pad, d_out, nb2, ba2)(g2, src2, dst2, zeros_o)
    z2 = _tc_post(n_pad, d_out, r)(part2, h2, dinv, b2)
    z2t = z2.T

    out = _decode_kernel(n_pad, nbd, ch)(z2t, src_d, dst_d, zeros_d)
    return out.reshape(-1)[:e2]


# speedup vs baseline: 1.0033x; 1.0033x over previous
import functools

import jax
import jax.numpy as jnp
from jax import lax
from jax.experimental import pallas as pl
from jax.experimental.pallas import tpu as pltpu
from jax.experimental.pallas import tpu_sc as plsc

NC = 2
NS = 16
NW = NC * NS


def _mesh():
    return plsc.VectorSubcoreMesh(core_axis_name="c", subcore_axis_name="s")


_SC_PARAMS = pltpu.CompilerParams(needs_layout_passes=False,
                                  use_tc_tiling_on_sc=False)


def _deg_kernel(n_pad, per_w):
    @functools.partial(
        pl.kernel,
        mesh=_mesh(),
        compiler_params=_SC_PARAMS,
        out_type=jax.ShapeDtypeStruct((NW, n_pad), jnp.float32),
        scratch_types=[
            pltpu.VMEM((per_w,), jnp.int32),
            pltpu.VMEM((n_pad,), jnp.float32),
        ],
    )
    def k(dst_hbm, zeros_hbm, deg_hbm, idx_v, hist_v):
        c = lax.axis_index("c")
        s = lax.axis_index("s")
        w = s * NC + c
        pltpu.sync_copy(dst_hbm.at[w], idx_v)
        pltpu.sync_copy(zeros_hbm.at[pl.ds(0, n_pad)], hist_v)
        ones = jnp.full((16,), 1.0, jnp.float32)

        def body(t, carry):
            v = idx_v[pl.ds(t * 16, 16)]
            plsc.addupdate_scatter(hist_v, [v], ones)
            return carry

        lax.fori_loop(0, per_w // 16, body, None)
        pltpu.sync_copy(hist_v, deg_hbm.at[w])

    return k


def _agg_kernel(n_pad, d, nb, b):
    rows = n_pad // NS

    @functools.partial(
        pl.kernel,
        mesh=_mesh(),
        compiler_params=_SC_PARAMS,
        out_type=jax.ShapeDtypeStruct((NC, n_pad, d), jnp.float32),
        scratch_types=[
            pltpu.VMEM((nb + 1, b), jnp.int32),
            pltpu.VMEM((nb, b), jnp.int32),
            pltpu.VMEM((b, d), jnp.float32),
            pltpu.VMEM_SHARED((n_pad, d), jnp.float32),
            pltpu.SemaphoreType.DMA,
        ],
    )
    def k(g_hbm, src_hbm, dst_hbm, zeros_hbm, out_hbm,
          src_v, dst_v, g0, acc_sh, sem0):
        c = lax.axis_index("c")
        s = lax.axis_index("s")
        w = s * NC + c
        pltpu.sync_copy(src_hbm.at[w], src_v)
        pltpu.sync_copy(dst_hbm.at[w], dst_v)
        pltpu.sync_copy(zeros_hbm, acc_sh.at[pl.ds(s * rows, rows)])
        plsc.subcore_barrier()

        def body(j, carry):
            pltpu.async_copy(g_hbm.at[src_v.at[j]], g0, sem0).wait()
            pltpu.sync_copy(g0, acc_sh.at[dst_v.at[j]], add=True)
            return carry

        lax.fori_loop(0, nb, body, None)
        plsc.subcore_barrier()
        pltpu.sync_copy(acc_sh.at[pl.ds(s * rows, rows)],
                        out_hbm.at[c, pl.ds(s * rows, rows)])

    return k


def _decode_kernel(n_pad, nbd, ch):
    rows_core = 2 * nbd
    rows_tile = rows_core // NS

    @functools.partial(
        pl.kernel,
        mesh=_mesh(),
        compiler_params=_SC_PARAMS,
        out_type=jax.ShapeDtypeStruct((4 * nbd, ch), jnp.float32),
        scratch_types=[
            pltpu.VMEM((8, n_pad), jnp.float32),
            pltpu.VMEM((ch,), jnp.int32),
            pltpu.VMEM((ch,), jnp.int32),
            pltpu.VMEM((ch,), jnp.int32),
            pltpu.VMEM((ch,), jnp.int32),
            pltpu.VMEM((16, ch), jnp.float32),
            pltpu.VMEM((16, ch), jnp.float32),
            pltpu.VMEM((16,), jnp.int32),
            pltpu.VMEM((16,), jnp.int32),
            pltpu.VMEM_SHARED((rows_core, ch), jnp.float32),
            pltpu.SemaphoreType.DMA,
            pltpu.SemaphoreType.DMA,
            pltpu.SemaphoreType.DMA,
            pltpu.SemaphoreType.DMA,
            pltpu.SemaphoreType.DMA,
            pltpu.SemaphoreType.DMA,
        ],
    )
    def k(zt_hbm, src_hbm, dst_hbm, zeros_hbm, out_hbm,
          zv, sv0, dv0, sv1, dv1, pv0, pv1, ri0, ri1, acc_sh,
          ss0, sd0, ss1, sd1, sa0, sa1):
        c = lax.axis_index("c")
        s = lax.axis_index("s")
        fg = lax.rem(s, 8)
        egl = s // 8
        eg = c * 2 + egl
        pltpu.sync_copy(zt_hbm.at[pl.ds(fg * 8, 8)], zv)
        pltpu.sync_copy(zeros_hbm,
                        acc_sh.at[pl.ds(s * rows_tile, rows_tile)])
        plsc.subcore_barrier()
        fidx = [jnp.full((16,), f, jnp.int32) for f in range(8)]
        pltpu.async_copy(src_hbm.at[eg, 0], sv0, ss0)
        pltpu.async_copy(dst_hbm.at[eg, 0], dv0, sd0)

        def process(sv, dv, pv, row):
            def block(off):
                s16 = sv[pl.ds(off, 16)]
                t16 = dv[pl.ds(off, 16)]
                acc0 = jnp.zeros((16,), jnp.float32)
                acc1 = jnp.zeros((16,), jnp.float32)
                for f in range(4):
                    a = plsc.load_gather(zv, [fidx[f], s16])
                    bb = plsc.load_gather(zv, [fidx[f], t16])
                    acc0 = acc0 + a * bb
                for f in range(4, 8):
                    a = plsc.load_gather(zv, [fidx[f], s16])
                    bb = plsc.load_gather(zv, [fidx[f], t16])
                    acc1 = acc1 + a * bb
                pv[row, pl.ds(off, 16)] = acc0 + acc1

            def inner(i, carry2):
                block(i * 16)
                return carry2

            lax.fori_loop(0, ch // 16, inner, None)

        def run_part(p, pv, ri, sa, drain_prev):
            if drain_prev:
                pltpu.make_async_copy(pv, acc_sh.at[ri], sa).wait()

            def chunk2(q2, carry1):
                kk = p * 16 + 2 * q2
                pltpu.async_copy(src_hbm.at[eg, kk + 1], sv1, ss1)
                pltpu.async_copy(dst_hbm.at[eg, kk + 1], dv1, sd1)
                pltpu.make_async_copy(src_hbm.at[eg, kk], sv0, ss0).wait()
                pltpu.make_async_copy(dst_hbm.at[eg, kk], dv0, sd0).wait()
                process(sv0, dv0, pv, 2 * q2)
                pltpu.async_copy(src_hbm.at[eg, kk + 2], sv0, ss0)
                pltpu.async_copy(dst_hbm.at[eg, kk + 2], dv0, sd0)
                pltpu.make_async_copy(src_hbm.at[eg, kk + 1], sv1, ss1).wait()
                pltpu.make_async_copy(dst_hbm.at[eg, kk + 1], dv1, sd1).wait()
                process(sv1, dv1, pv, 2 * q2 + 1)
                return carry1

            lax.fori_loop(0, 8, chunk2, None)
            ri[...] = lax.iota(jnp.int32, 16) + (egl * nbd + p * 16)
            pltpu.async_copy(pv, acc_sh.at[ri], sa, add=True)

        run_part(0, pv0, ri0, sa0, False)
        run_part(1, pv1, ri1, sa1, False)

        def part2(t, carry):
            run_part(2 * t, pv0, ri0, sa0, True)
            run_part(2 * t + 1, pv1, ri1, sa1, True)
            return carry

        lax.fori_loop(1, nbd // 32, part2, None)
        pltpu.make_async_copy(src_hbm.at[eg, nbd], sv0, ss0).wait()
        pltpu.make_async_copy(dst_hbm.at[eg, nbd], dv0, sd0).wait()
        pltpu.make_async_copy(pv0, acc_sh.at[ri0], sa0).wait()
        pltpu.make_async_copy(pv1, acc_sh.at[ri1], sa1).wait()
        plsc.subcore_barrier()
        pltpu.sync_copy(acc_sh.at[pl.ds(s * rows_tile, rows_tile)],
                        out_hbm.at[pl.ds(c * rows_core + s * rows_tile, rows_tile)])

    return k


def _tc_prep(n_pad, d_in, d_h, r):
    def body(deg2_ref, mask_ref, x_ref, w1_ref, h1_ref, g1_ref, dinv_ref):
        deg = jnp.sum(deg2_ref[...], axis=0) + mask_ref[...]
        dinv = jnp.where(deg > 0, lax.rsqrt(deg), 0.0)
        h1 = jnp.dot(x_ref[...], w1_ref[...], preferred_element_type=jnp.float32)
        h1_ref[...] = h1
        g1_ref[...] = h1 * dinv[:, None]
        dinv_ref[...] = dinv

    return pl.pallas_call(
        body,
        grid=(n_pad // r,),
        in_specs=[
            pl.BlockSpec((NW, r), lambda i: (0, i)),
            pl.BlockSpec((r,), lambda i: (i,)),
            pl.BlockSpec((r, d_in), lambda i: (i, 0)),
            pl.BlockSpec((d_in, d_h), lambda i: (0, 0)),
        ],
        out_specs=[
            pl.BlockSpec((r, d_h), lambda i: (i, 0)),
            pl.BlockSpec((r, d_h), lambda i: (i, 0)),
            pl.BlockSpec((r,), lambda i: (i,)),
        ],
        out_shape=[
            jax.ShapeDtypeStruct((n_pad, d_h), jnp.float32),
            jax.ShapeDtypeStruct((n_pad, d_h), jnp.float32),
            jax.ShapeDtypeStruct((n_pad,), jnp.float32),
        ],
    )


def _tc_mid(n_pad, d_h, d_out, r):
    def body(p_ref, h1_ref, dinv_ref, b1_ref, w2_ref, h2_ref, g2_ref):
        dinv = dinv_ref[...]
        z1 = (dinv[:, None] * (p_ref[0] + p_ref[1])
              + (dinv * dinv)[:, None] * h1_ref[...] + b1_ref[...][None, :])
        z1 = jnp.maximum(z1, 0.0)
        h2 = jnp.dot(z1, w2_ref[...], preferred_element_type=jnp.float32)
        h2_ref[...] = h2
        g2_ref[...] = h2 * dinv[:, None]

    return pl.pallas_call(
        body,
        grid=(n_pad // r,),
        in_specs=[
            pl.BlockSpec((2, r, d_h), lambda i: (0, i, 0)),
            pl.BlockSpec((r, d_h), lambda i: (i, 0)),
            pl.BlockSpec((r,), lambda i: (i,)),
            pl.BlockSpec((d_h,), lambda i: (0,)),
            pl.BlockSpec((d_h, d_out), lambda i: (0, 0)),
        ],
        out_specs=[
            pl.BlockSpec((r, d_out), lambda i: (i, 0)),
            pl.BlockSpec((r, d_out), lambda i: (i, 0)),
        ],
        out_shape=[
            jax.ShapeDtypeStruct((n_pad, d_out), jnp.float32),
            jax.ShapeDtypeStruct((n_pad, d_out), jnp.float32),
        ],
    )


def _tc_post(n_pad, d_out, r):
    def body(p_ref, h2_ref, dinv_ref, b2_ref, z2_ref):
        dinv = dinv_ref[...]
        z2_ref[...] = (dinv[:, None] * (p_ref[0] + p_ref[1])
                       + (dinv * dinv)[:, None] * h2_ref[...]
                       + b2_ref[...][None, :])

    return pl.pallas_call(
        body,
        grid=(n_pad // r,),
        in_specs=[
            pl.BlockSpec((2, r, d_out), lambda i: (0, i, 0)),
            pl.BlockSpec((r, d_out), lambda i: (i, 0)),
            pl.BlockSpec((r,), lambda i: (i,)),
            pl.BlockSpec((d_out,), lambda i: (0,)),
        ],
        out_specs=pl.BlockSpec((r, d_out), lambda i: (i, 0)),
        out_shape=jax.ShapeDtypeStruct((n_pad, d_out), jnp.float32),
    )


def kernel(x, pos_edge_index, neg_edge_index, W1, b1, W2, b2):
    n, d_in = x.shape
    d_h = W1.shape[1]
    d_out = W2.shape[1]
    e = pos_edge_index.shape[1]

    n_pad = ((n + NW * 8 - 1) // (NW * 8)) * (NW * 8)
    dummy = n

    spare = max(n_pad - n, 1)

    def spread_pad(idx, e_pad):
        pad = dummy + (jnp.arange(e_pad - e, dtype=idx.dtype) % spare)
        return jnp.concatenate([idx, pad])

    def agg_layout(b):
        nb = -(-e // (NW * b))
        nb = nb + (nb % 2)
        e_pad = NW * nb * b
        sp = spread_pad(pos_edge_index[0], e_pad).reshape(NW, nb, b)
        dp = spread_pad(pos_edge_index[1], e_pad).reshape(NW, nb, b)
        sp = jnp.concatenate(
            [sp, jnp.full((NW, 1, b), dummy, sp.dtype)], axis=1)
        return sp, dp, nb

    ba1 = 128
    ba2 = 128
    src1, dst1, nb1 = agg_layout(ba1)
    src2, dst2, nb2 = agg_layout(ba2)
    dst_flat = dst2.reshape(NW, nb2 * ba2)

    ch = 512
    e2 = 2 * e
    nbd = -(-e2 // (4 * ch))
    nbd = ((nbd + 31) // 32) * 32
    e2_pad = 4 * nbd * ch
    pad_d = dummy + (jnp.arange(e2_pad - e2, dtype=jnp.int32) % spare)
    src_d = jnp.concatenate(
        [pos_edge_index[0], neg_edge_index[0], pad_d]).reshape(4, nbd, ch)
    dst_d = jnp.concatenate(
        [pos_edge_index[1], neg_edge_index[1], pad_d]).reshape(4, nbd, ch)
    pad_chunk = jnp.full((4, 1, ch), dummy, jnp.int32)
    src_d = jnp.concatenate([src_d, pad_chunk], axis=1)
    dst_d = jnp.concatenate([dst_d, pad_chunk], axis=1)

    xp = jnp.zeros((n_pad, d_in), jnp.float32).at[:n].set(x)
    maskf = (jnp.arange(n_pad) < n).astype(jnp.float32)

    zeros1d = jnp.zeros((n_pad,), jnp.float32)
    zeros_h = jnp.zeros((n_pad // NS, d_h), jnp.float32)
    zeros_o = jnp.zeros((n_pad // NS, d_out), jnp.float32)
    zeros_d = jnp.zeros(((2 * nbd) // NS, ch), jnp.float32)

    r = 1024

    deg2 = _deg_kernel(n_pad, nb2 * ba2)(dst_flat, zeros1d)
    h1, g1, dinv = _tc_prep(n_pad, d_in, d_h, r)(deg2, maskf, xp, W1)
    part1 = _agg_kernel(n_pad, d_h, nb1, ba1)(g1, src1, dst1, zeros_h)
    h2, g2 = _tc_mid(n_pad, d_h, d_out, r)(part1, h1, dinv, b1, W2)
    part2 = _agg_kernel(n_pad, d_out, nb2, ba2)(g2, src2, dst2, zeros_o)
    z2 = _tc_post(n_pad, d_out, r)(part2, h2, dinv, b2)
    z2t = z2.T

    out = _decode_kernel(n_pad, nbd, ch)(z2t, src_d, dst_d, zeros_d)
    return out.reshape(-1)[:e2]
